# baseline (speedup 1.0000x reference)
import functools

import jax
import jax.numpy as jnp
from jax import lax
from jax.experimental import pallas as pl
from jax.experimental.pallas import tpu as pltpu

HP = 256
GEMM_CHUNK = 16
LSTM_UNROLL = 8
VIT_UNROLL = 8


def _lstm_cell(g_in, whh, h_prev, c_prev, m):
    gates = g_in + jnp.dot(h_prev.astype(jnp.bfloat16), whh,
                           preferred_element_type=jnp.float32)
    ig = jax.nn.sigmoid(gates[:, 0 * HP:1 * HP])
    fg = jax.nn.sigmoid(gates[:, 1 * HP:2 * HP])
    gg = jnp.tanh(gates[:, 2 * HP:3 * HP])
    og = jax.nn.sigmoid(gates[:, 3 * HP:4 * HP])
    c_new = fg * c_prev + ig * gg
    h_new = og * jnp.tanh(c_new)
    c_out = jnp.where(m, c_new, c_prev)
    h_out = jnp.where(m, h_new, h_prev)
    return h_new, h_out, c_out


def _inproj_half(src_ref, w_ref, b_ref, g_scr, phase, t_total, bp):
    din = src_ref.shape[-1]
    th = t_total // 2
    for d in range(2):
        w_in = w_ref[d]
        bias = b_ref[d]
        base = (phase if d == 0 else 1 - phase) * th
        for i in range(th // GEMM_CHUNK):
            xb = src_ref[base + i * GEMM_CHUNK:base + (i + 1) * GEMM_CHUNK]
            x2d = xb.reshape(GEMM_CHUNK * bp, din).astype(jnp.bfloat16)
            acc = jnp.dot(x2d, w_in,
                          preferred_element_type=jnp.float32) + bias
            g_scr[d, i * GEMM_CHUNK:(i + 1) * GEMM_CHUNK] = (
                acc.astype(jnp.bfloat16).reshape(GEMM_CHUNK, bp, 4 * HP))


def _bilstm_layer(src_ref, w_ref, b_ref, lenf, whh_ref, out_scr, g_scr,
                  hf_scr, cf_scr, hb_scr, cb_scr, t_total, bp, unroll):
    hf_scr[...] = jnp.zeros_like(hf_scr)
    cf_scr[...] = jnp.zeros_like(cf_scr)
    hb_scr[...] = jnp.zeros_like(hb_scr)
    cb_scr[...] = jnp.zeros_like(cb_scr)
    whh_f = whh_ref[0]
    whh_b = whh_ref[1]
    th = t_total // 2

    for phase in range(2):
        _inproj_half(src_ref, w_ref, b_ref, g_scr, phase, t_total, bp)
        f_base = phase * th
        b_base = (1 - phase) * th

        def step(s, carry):
            tf = s
            tb = t_total - 1 - s
            g_f = g_scr[0, tf - f_base].astype(jnp.float32)
            g_b = g_scr[1, tb - b_base].astype(jnp.float32)
            mf = lenf > tf
            mb = lenf > tb
            hf_new, hf_out, cf_out = _lstm_cell(g_f, whh_f, hf_scr[...],
                                                cf_scr[...], mf)
            hb_new, hb_out, cb_out = _lstm_cell(g_b, whh_b, hb_scr[...],
                                                cb_scr[...], mb)
            hf_scr[...] = hf_out
            cf_scr[...] = cf_out
            hb_scr[...] = hb_out
            cb_scr[...] = cb_out
            out_scr[tf, :, 0:HP] = jnp.where(mf, hf_new, 0.0).astype(out_scr.dtype)
            out_scr[tb, :, HP:2 * HP] = jnp.where(mb, hb_new, 0.0).astype(out_scr.dtype)
            return carry

        lax.fori_loop(phase * th, (phase + 1) * th, step, 0, unroll=unroll)


def _net_kernel(x_ref, len_ref,
                w0_ref, b0_ref, whh0_ref,
                w1_ref, b1_ref, whh1_ref,
                wout_ref, bout_ref, start_ref, end_ref, trans_ref,
                tags_ref,
                g_scr, h0_scr, h1_scr, em_scr, hist_scr,
                hf_scr, cf_scr, hb_scr, cb_scr,
                *, t_total, unroll, vunroll):
    bp = x_ref.shape[1]
    lenf = len_ref[...]

    _bilstm_layer(x_ref, w0_ref, b0_ref, lenf, whh0_ref, h0_scr, g_scr,
                  hf_scr, cf_scr, hb_scr, cb_scr, t_total, bp, unroll)
    _bilstm_layer(h0_scr, w1_ref, b1_ref, lenf, whh1_ref, h1_scr, g_scr,
                  hf_scr, cf_scr, hb_scr, cb_scr, t_total, bp, unroll)

    d2 = h1_scr.shape[-1]
    k8, kp = trans_ref.shape
    for i in range(t_total // GEMM_CHUNK):
        hb = h1_scr[i * GEMM_CHUNK:(i + 1) * GEMM_CHUNK]
        em = (jnp.dot(hb.reshape(GEMM_CHUNK * bp, d2), wout_ref[...],
                      preferred_element_type=jnp.float32) + bout_ref[...])
        em_scr[i * GEMM_CHUNK:(i + 1) * GEMM_CHUNK] = (
            em.reshape(GEMM_CHUNK, bp, kp))

    trans8 = trans_ref[...]
    idx8 = lax.broadcasted_iota(jnp.int32, (bp, k8, kp), 1).astype(jnp.float32)

    def fstep(t, score):
        em = em_scr[t]
        m = lenf > t
        prev = score
        prev8 = prev[:, :k8]
        cand = prev8[:, :, None] + trans8[None, :, :]
        best = jnp.max(cand, axis=1)
        is_best = cand >= best[:, None, :]
        bidx = jnp.min(jnp.where(is_best, idx8, float(k8)), axis=1)
        upd = jnp.where(m, best + em, prev)
        nxt = jnp.where(t == 0, prev + em, upd)
        hist_scr[t] = jnp.where(t == 0, 0.0, bidx).astype(jnp.bfloat16)
        return nxt

    score0 = jnp.broadcast_to(start_ref[...], (bp, kp))
    score = lax.fori_loop(0, t_total, fstep, score0, unroll=vunroll)
    score = score + end_ref[...]

    lane = lax.broadcasted_iota(jnp.int32, (bp, kp), 1).astype(jnp.float32)
    maxv = jnp.max(score, axis=1, keepdims=True)
    best_last = jnp.min(jnp.where(score == maxv, lane, float(kp)),
                        axis=1, keepdims=True)
    seq_end = lenf - 1.0
    tlane = lax.broadcasted_iota(jnp.int32, (bp, t_total), 1).astype(jnp.float32)

    def bstep(s, carry):
        tags_acc, cur = carry
        t = t_total - 1 - s
        h = hist_scr[jnp.minimum(t + 1, t_total - 1)].astype(jnp.float32)
        picked = jnp.sum(jnp.where(lane == cur, h, 0.0), axis=1, keepdims=True)
        tag_t = jnp.where(t == seq_end, best_last,
                          jnp.where(t < seq_end, picked, 0.0))
        cur = jnp.where(t <= seq_end, tag_t, cur)
        tags_acc = jnp.where(tlane == t, tag_t, tags_acc)
        return (tags_acc, cur)

    tags0 = jnp.zeros((bp, t_total), jnp.float32)
    tags_acc, _ = lax.fori_loop(0, t_total, bstep, (tags0, best_last),
                                unroll=vunroll)
    tags_ref[...] = jnp.transpose(tags_acc.astype(jnp.int32))


def kernel(pad_index, embedding, w_out, b_out, crf_start, crf_end, crf_trans,
           layer0_w_in, layer0_b_in, layer0_whh,
           layer1_w_in, layer1_b_in, layer1_whh,
           x_ids, sent_lengths):
    t_total, b_full = x_ids.shape
    k8, kp = crf_trans.shape
    len_col = sent_lengths.astype(jnp.float32)[:, None]
    x_emb = embedding[x_ids.astype(jnp.int32)]

    def stack2(w):
        return jnp.stack([w[:, :4 * HP], w[:, 4 * HP:]])

    kern = functools.partial(_net_kernel, t_total=t_total,
                             unroll=LSTM_UNROLL, vunroll=VIT_UNROLL)
    whole = lambda shape: pl.BlockSpec(shape, lambda i: (0,) * len(shape))
    emb_dim = x_emb.shape[-1]
    tags = pl.pallas_call(
        kern,
        out_shape=jax.ShapeDtypeStruct((t_total, b_full), jnp.int32),
        grid_spec=pltpu.PrefetchScalarGridSpec(
            num_scalar_prefetch=0,
            grid=(1,),
            in_specs=[
                whole((t_total, b_full, emb_dim)),
                whole((b_full, 1)),
                whole((2, emb_dim, 4 * HP)),
                whole((2, 1, 4 * HP)),
                whole((2, HP, 4 * HP)),
                whole((2, 2 * HP, 4 * HP)),
                whole((2, 1, 4 * HP)),
                whole((2, HP, 4 * HP)),
                whole((2 * HP, kp)),
                whole((1, kp)),
                whole((1, kp)),
                whole((1, kp)),
                whole((k8, kp)),
            ],
            out_specs=whole((t_total, b_full)),
            scratch_shapes=[
                pltpu.VMEM((2, t_total // 2, b_full, 4 * HP), jnp.bfloat16),
                pltpu.VMEM((t_total, b_full, 2 * HP), jnp.bfloat16),
                pltpu.VMEM((t_total, b_full, 2 * HP), jnp.bfloat16),
                pltpu.VMEM((t_total, b_full, kp), jnp.float32),
                pltpu.VMEM((t_total, b_full, kp), jnp.bfloat16),
                pltpu.VMEM((b_full, HP), jnp.float32),
                pltpu.VMEM((b_full, HP), jnp.float32),
                pltpu.VMEM((b_full, HP), jnp.float32),
                pltpu.VMEM((b_full, HP), jnp.float32),
            ],
        ),
        compiler_params=pltpu.CompilerParams(
            dimension_semantics=("arbitrary",)),
    )(x_emb, len_col,
      stack2(layer0_w_in), stack2(layer0_b_in), layer0_whh,
      stack2(layer1_w_in), stack2(layer1_b_in), layer1_whh,
      w_out, b_out, crf_start, crf_end, crf_trans)
    return tags


# speedup vs baseline: 1.0000x; 1.0000x over previous
import functools

import jax
import jax.numpy as jnp
from jax import lax
from jax.experimental import pallas as pl
from jax.experimental.pallas import tpu as pltpu

HP = 256
GEMM_CHUNK = 16
LSTM_UNROLL = 8
VIT_UNROLL = 8


def _lstm_cell(g_in, whh, h_prev, c_prev, m):
    gates = g_in + jnp.dot(h_prev.astype(jnp.bfloat16), whh,
                           preferred_element_type=jnp.float32)
    ig = jax.nn.sigmoid(gates[:, 0 * HP:1 * HP])
    fg = jax.nn.sigmoid(gates[:, 1 * HP:2 * HP])
    gg = jnp.tanh(gates[:, 2 * HP:3 * HP])
    og = jax.nn.sigmoid(gates[:, 3 * HP:4 * HP])
    c_new = fg * c_prev + ig * gg
    h_new = og * jnp.tanh(c_new)
    c_out = jnp.where(m, c_new, c_prev)
    h_out = jnp.where(m, h_new, h_prev)
    return h_new, h_out, c_out


def _inproj_half(src_ref, w_ref, b_ref, g_scr, phase, t_total, bp):
    din = src_ref.shape[-1]
    th = t_total // 2
    for d in range(2):
        w_in = w_ref[d]
        bias = b_ref[d]
        base = (phase if d == 0 else 1 - phase) * th
        for i in range(th // GEMM_CHUNK):
            xb = src_ref[base + i * GEMM_CHUNK:base + (i + 1) * GEMM_CHUNK]
            x2d = xb.reshape(GEMM_CHUNK * bp, din).astype(jnp.bfloat16)
            acc = jnp.dot(x2d, w_in,
                          preferred_element_type=jnp.float32) + bias
            g_scr[d, i * GEMM_CHUNK:(i + 1) * GEMM_CHUNK] = (
                acc.astype(jnp.bfloat16).reshape(GEMM_CHUNK, bp, 4 * HP))


def _bilstm_layer(src_ref, w_ref, b_ref, lenf, whh_ref, out_scr, g_scr,
                  hf_scr, cf_scr, hb_scr, cb_scr, t_total, bp, unroll):
    hf_scr[...] = jnp.zeros_like(hf_scr)
    cf_scr[...] = jnp.zeros_like(cf_scr)
    hb_scr[...] = jnp.zeros_like(hb_scr)
    cb_scr[...] = jnp.zeros_like(cb_scr)
    whh_f = whh_ref[0]
    whh_b = whh_ref[1]
    th = t_total // 2

    for phase in range(2):
        _inproj_half(src_ref, w_ref, b_ref, g_scr, phase, t_total, bp)
        f_base = phase * th
        b_base = (1 - phase) * th

        def step(s, carry):
            tf = s
            tb = t_total - 1 - s
            g_f = g_scr[0, tf - f_base].astype(jnp.float32)
            g_b = g_scr[1, tb - b_base].astype(jnp.float32)
            mf = lenf > tf
            mb = lenf > tb
            hf_new, hf_out, cf_out = _lstm_cell(g_f, whh_f, hf_scr[...],
                                                cf_scr[...], mf)
            hb_new, hb_out, cb_out = _lstm_cell(g_b, whh_b, hb_scr[...],
                                                cb_scr[...], mb)
            hf_scr[...] = hf_out
            cf_scr[...] = cf_out
            hb_scr[...] = hb_out
            cb_scr[...] = cb_out
            out_scr[tf, :, 0:HP] = jnp.where(mf, hf_new, 0.0).astype(out_scr.dtype)
            out_scr[tb, :, HP:2 * HP] = jnp.where(mb, hb_new, 0.0).astype(out_scr.dtype)
            return carry

        lax.fori_loop(phase * th, (phase + 1) * th, step, 0, unroll=unroll)


def _net_kernel(x_ref, len_ref,
                w0_ref, b0_ref, whh0_ref,
                w1_ref, b1_ref, whh1_ref,
                wout_ref, bout_ref, start_ref, end_ref, trans_ref,
                tags_ref,
                g_scr, h0_scr, h1_scr, em_scr, hist_scr,
                hf_scr, cf_scr, hb_scr, cb_scr,
                *, t_total, unroll, vunroll):
    bp = x_ref.shape[1]
    lenf = len_ref[...]

    _bilstm_layer(x_ref, w0_ref, b0_ref, lenf, whh0_ref, h0_scr, g_scr,
                  hf_scr, cf_scr, hb_scr, cb_scr, t_total, bp, unroll)
    _bilstm_layer(h0_scr, w1_ref, b1_ref, lenf, whh1_ref, h1_scr, g_scr,
                  hf_scr, cf_scr, hb_scr, cb_scr, t_total, bp, unroll)

    d2 = h1_scr.shape[-1]
    k8, kp = trans_ref.shape
    for i in range(t_total // GEMM_CHUNK):
        hb = h1_scr[i * GEMM_CHUNK:(i + 1) * GEMM_CHUNK]
        em = (jnp.dot(hb.reshape(GEMM_CHUNK * bp, d2), wout_ref[...],
                      preferred_element_type=jnp.float32) + bout_ref[...])
        em_scr[i * GEMM_CHUNK:(i + 1) * GEMM_CHUNK] = (
            em.reshape(GEMM_CHUNK, bp, kp))

    trans8 = trans_ref[...]
    idx8 = lax.broadcasted_iota(jnp.int32, (bp, k8, kp), 1).astype(jnp.float32)

    def fstep(t, score):
        em = em_scr[t]
        m = lenf > t
        prev = score
        prev8 = prev[:, :k8]
        cand = prev8[:, :, None] + trans8[None, :, :]
        best = jnp.max(cand, axis=1)
        is_best = cand >= best[:, None, :]
        bidx = jnp.min(jnp.where(is_best, idx8, float(k8)), axis=1)
        upd = jnp.where(m, best + em, prev)
        nxt = jnp.where(t == 0, prev + em, upd)
        hist_scr[t] = jnp.where(t == 0, 0.0, bidx).astype(jnp.bfloat16)
        return nxt

    score0 = jnp.broadcast_to(start_ref[...], (bp, kp))
    score = lax.fori_loop(0, t_total, fstep, score0, unroll=vunroll)
    score = score + end_ref[...]

    lane = lax.broadcasted_iota(jnp.int32, (bp, kp), 1).astype(jnp.float32)
    maxv = jnp.max(score, axis=1, keepdims=True)
    best_last = jnp.min(jnp.where(score == maxv, lane, float(kp)),
                        axis=1, keepdims=True)
    seq_end = lenf - 1.0
    tlane = lax.broadcasted_iota(jnp.int32, (bp, t_total), 1).astype(jnp.float32)

    def bstep(s, carry):
        tags_acc, cur = carry
        t = t_total - 1 - s
        h = hist_scr[jnp.minimum(t + 1, t_total - 1)].astype(jnp.float32)
        picked = jnp.sum(jnp.where(lane == cur, h, 0.0), axis=1, keepdims=True)
        tag_t = jnp.where(t == seq_end, best_last,
                          jnp.where(t < seq_end, picked, 0.0))
        cur = jnp.where(t <= seq_end, tag_t, cur)
        tags_acc = jnp.where(tlane == t, tag_t, tags_acc)
        return (tags_acc, cur)

    tags0 = jnp.zeros((bp, t_total), jnp.float32)
    tags_acc, _ = lax.fori_loop(0, t_total, bstep, (tags0, best_last),
                                unroll=vunroll)
    tags_ref[...] = tags_acc.astype(jnp.int32)


def kernel(pad_index, embedding, w_out, b_out, crf_start, crf_end, crf_trans,
           layer0_w_in, layer0_b_in, layer0_whh,
           layer1_w_in, layer1_b_in, layer1_whh,
           x_ids, sent_lengths):
    t_total, b_full = x_ids.shape
    k8, kp = crf_trans.shape
    len_col = sent_lengths.astype(jnp.float32)[:, None]
    x_emb = embedding[x_ids.astype(jnp.int32)]

    def stack2(w):
        return jnp.stack([w[:, :4 * HP], w[:, 4 * HP:]])

    kern = functools.partial(_net_kernel, t_total=t_total,
                             unroll=LSTM_UNROLL, vunroll=VIT_UNROLL)
    whole = lambda shape: pl.BlockSpec(shape, lambda i: (0,) * len(shape))
    emb_dim = x_emb.shape[-1]
    tags_bt = pl.pallas_call(
        kern,
        out_shape=jax.ShapeDtypeStruct((b_full, t_total), jnp.int32),
        grid_spec=pltpu.PrefetchScalarGridSpec(
            num_scalar_prefetch=0,
            grid=(1,),
            in_specs=[
                whole((t_total, b_full, emb_dim)),
                whole((b_full, 1)),
                whole((2, emb_dim, 4 * HP)),
                whole((2, 1, 4 * HP)),
                whole((2, HP, 4 * HP)),
                whole((2, 2 * HP, 4 * HP)),
                whole((2, 1, 4 * HP)),
                whole((2, HP, 4 * HP)),
                whole((2 * HP, kp)),
                whole((1, kp)),
                whole((1, kp)),
                whole((1, kp)),
                whole((k8, kp)),
            ],
            out_specs=whole((b_full, t_total)),
            scratch_shapes=[
                pltpu.VMEM((2, t_total // 2, b_full, 4 * HP), jnp.bfloat16),
                pltpu.VMEM((t_total, b_full, 2 * HP), jnp.bfloat16),
                pltpu.VMEM((t_total, b_full, 2 * HP), jnp.bfloat16),
                pltpu.VMEM((t_total, b_full, kp), jnp.float32),
                pltpu.VMEM((t_total, b_full, kp), jnp.bfloat16),
                pltpu.VMEM((b_full, HP), jnp.float32),
                pltpu.VMEM((b_full, HP), jnp.float32),
                pltpu.VMEM((b_full, HP), jnp.float32),
                pltpu.VMEM((b_full, HP), jnp.float32),
            ],
        ),
        compiler_params=pltpu.CompilerParams(
            dimension_semantics=("arbitrary",)),
    )(x_emb, len_col,
      stack2(layer0_w_in), stack2(layer0_b_in), layer0_whh,
      stack2(layer1_w_in), stack2(layer1_b_in), layer1_whh,
      w_out, b_out, crf_start, crf_end, crf_trans)
    return tags_bt.T


# speedup vs baseline: 1.8086x; 1.8086x over previous
import functools

import jax
import jax.numpy as jnp
from jax import lax
from jax.experimental import pallas as pl
from jax.experimental.pallas import tpu as pltpu

HP = 256
GEMM_CHUNK = 16
LSTM_UNROLL = 8
VIT_UNROLL = 8


def _lstm_cell(g_in, whh, h_prev, c_prev, m):
    gates = g_in + jnp.dot(h_prev.astype(jnp.bfloat16), whh,
                           preferred_element_type=jnp.float32)
    ig = jax.nn.sigmoid(gates[:, 0 * HP:1 * HP])
    fg = jax.nn.sigmoid(gates[:, 1 * HP:2 * HP])
    gg = jnp.tanh(gates[:, 2 * HP:3 * HP])
    og = jax.nn.sigmoid(gates[:, 3 * HP:4 * HP])
    c_new = fg * c_prev + ig * gg
    h_new = og * jnp.tanh(c_new)
    c_out = jnp.where(m, c_new, c_prev)
    h_out = jnp.where(m, h_new, h_prev)
    return h_new, h_out, c_out


def _inproj_half(src_ref, w_ref, b_ref, g_scr, phase, t_total, bp):
    din = src_ref.shape[-1]
    th = t_total // 2
    for d in range(2):
        w_in = w_ref[d]
        bias = b_ref[d]
        base = (phase if d == 0 else 1 - phase) * th
        for i in range(th // GEMM_CHUNK):
            xb = src_ref[base + i * GEMM_CHUNK:base + (i + 1) * GEMM_CHUNK]
            x2d = xb.reshape(GEMM_CHUNK * bp, din).astype(jnp.bfloat16)
            acc = jnp.dot(x2d, w_in,
                          preferred_element_type=jnp.float32) + bias
            g_scr[d, i * GEMM_CHUNK:(i + 1) * GEMM_CHUNK] = (
                acc.astype(jnp.bfloat16).reshape(GEMM_CHUNK, bp, 4 * HP))


def _bilstm_layer(src_ref, w_ref, b_ref, lenf, whh_ref, out_scr, g_scr,
                  hf_scr, cf_scr, hb_scr, cb_scr, t_total, bp, unroll):
    hf_scr[...] = jnp.zeros_like(hf_scr)
    cf_scr[...] = jnp.zeros_like(cf_scr)
    hb_scr[...] = jnp.zeros_like(hb_scr)
    cb_scr[...] = jnp.zeros_like(cb_scr)
    whh_f = whh_ref[0]
    whh_b = whh_ref[1]
    th = t_total // 2

    for phase in range(2):
        _inproj_half(src_ref, w_ref, b_ref, g_scr, phase, t_total, bp)
        f_base = phase * th
        b_base = (1 - phase) * th

        def step(s, carry):
            tf = s
            tb = t_total - 1 - s
            g_f = g_scr[0, tf - f_base].astype(jnp.float32)
            g_b = g_scr[1, tb - b_base].astype(jnp.float32)
            mf = lenf > tf
            mb = lenf > tb
            hf_new, hf_out, cf_out = _lstm_cell(g_f, whh_f, hf_scr[...],
                                                cf_scr[...], mf)
            hb_new, hb_out, cb_out = _lstm_cell(g_b, whh_b, hb_scr[...],
                                                cb_scr[...], mb)
            hf_scr[...] = hf_out
            cf_scr[...] = cf_out
            hb_scr[...] = hb_out
            cb_scr[...] = cb_out
            out_scr[tf, :, 0:HP] = jnp.where(mf, hf_new, 0.0).astype(out_scr.dtype)
            out_scr[tb, :, HP:2 * HP] = jnp.where(mb, hb_new, 0.0).astype(out_scr.dtype)
            return carry

        lax.fori_loop(phase * th, (phase + 1) * th, step, 0, unroll=unroll)


def _net_kernel(x_ref, len_ref,
                w0_ref, b0_ref, whh0_ref,
                w1_ref, b1_ref, whh1_ref,
                wout_ref, bout_ref, start_ref, end_ref, trans_ref,
                tags_ref,
                g_scr, h0_scr, h1_scr, em_scr, hist_scr,
                hf_scr, cf_scr, hb_scr, cb_scr,
                *, t_total, unroll, vunroll):
    bp = x_ref.shape[1]
    lenf = len_ref[...]

    _bilstm_layer(x_ref, w0_ref, b0_ref, lenf, whh0_ref, h0_scr, g_scr,
                  hf_scr, cf_scr, hb_scr, cb_scr, t_total, bp, unroll)
    _bilstm_layer(h0_scr, w1_ref, b1_ref, lenf, whh1_ref, h1_scr, g_scr,
                  hf_scr, cf_scr, hb_scr, cb_scr, t_total, bp, unroll)

    d2 = h1_scr.shape[-1]
    k8, kp = trans_ref.shape
    for i in range(t_total // GEMM_CHUNK):
        hb = h1_scr[i * GEMM_CHUNK:(i + 1) * GEMM_CHUNK]
        em = (jnp.dot(hb.reshape(GEMM_CHUNK * bp, d2), wout_ref[...],
                      preferred_element_type=jnp.float32) + bout_ref[...])
        em_scr[i * GEMM_CHUNK:(i + 1) * GEMM_CHUNK] = (
            em.reshape(GEMM_CHUNK, bp, kp))

    trans8 = trans_ref[...]
    idx8 = lax.broadcasted_iota(jnp.int32, (bp, k8, kp), 1).astype(jnp.float32)

    def fstep(t, score):
        em = em_scr[t]
        m = lenf > t
        prev = score
        prev8 = prev[:, :k8]
        cand = prev8[:, :, None] + trans8[None, :, :]
        best = jnp.max(cand, axis=1)
        is_best = cand >= best[:, None, :]
        bidx = jnp.min(jnp.where(is_best, idx8, float(k8)), axis=1)
        upd = jnp.where(m, best + em, prev)
        nxt = jnp.where(t == 0, prev + em, upd)
        hist_scr[t] = jnp.where(t == 0, 0.0, bidx).astype(jnp.bfloat16)
        return nxt

    score0 = jnp.broadcast_to(start_ref[...], (bp, kp))
    score = lax.fori_loop(0, t_total, fstep, score0, unroll=vunroll)
    score = score + end_ref[...]

    lane = lax.broadcasted_iota(jnp.int32, (bp, kp), 1).astype(jnp.float32)
    maxv = jnp.max(score, axis=1, keepdims=True)
    best_last = jnp.min(jnp.where(score == maxv, lane, float(kp)),
                        axis=1, keepdims=True)
    seq_end = lenf - 1.0
    tlane = lax.broadcasted_iota(jnp.int32, (bp, t_total), 1).astype(jnp.float32)

    def bstep(s, carry):
        tags_acc, cur = carry
        t = t_total - 1 - s
        h = hist_scr[jnp.minimum(t + 1, t_total - 1)].astype(jnp.float32)
        picked = jnp.sum(jnp.where(lane == cur, h, 0.0), axis=1, keepdims=True)
        tag_t = jnp.where(t == seq_end, best_last,
                          jnp.where(t < seq_end, picked, 0.0))
        cur = jnp.where(t <= seq_end, tag_t, cur)
        tags_acc = jnp.where(tlane == t, tag_t, tags_acc)
        return (tags_acc, cur)

    tags0 = jnp.zeros((bp, t_total), jnp.float32)
    tags_acc, _ = lax.fori_loop(0, t_total, bstep, (tags0, best_last),
                                unroll=vunroll)
    tags_ref[...] = tags_acc.astype(jnp.int32)


def kernel(pad_index, embedding, w_out, b_out, crf_start, crf_end, crf_trans,
           layer0_w_in, layer0_b_in, layer0_whh,
           layer1_w_in, layer1_b_in, layer1_whh,
           x_ids, sent_lengths):
    t_total, b_full = x_ids.shape
    k8, kp = crf_trans.shape
    len_col = sent_lengths.astype(jnp.float32)[:, None]
    x_emb = embedding[x_ids.astype(jnp.int32)].astype(jnp.bfloat16)

    def stack2(w):
        return jnp.stack([w[:, :4 * HP], w[:, 4 * HP:]])

    kern = functools.partial(_net_kernel, t_total=t_total,
                             unroll=LSTM_UNROLL, vunroll=VIT_UNROLL)
    whole = lambda shape: pl.BlockSpec(shape, lambda i: (0,) * len(shape))
    emb_dim = x_emb.shape[-1]
    tags_bt = pl.pallas_call(
        kern,
        out_shape=jax.ShapeDtypeStruct((b_full, t_total), jnp.int32),
        grid_spec=pltpu.PrefetchScalarGridSpec(
            num_scalar_prefetch=0,
            grid=(1,),
            in_specs=[
                whole((t_total, b_full, emb_dim)),
                whole((b_full, 1)),
                whole((2, emb_dim, 4 * HP)),
                whole((2, 1, 4 * HP)),
                whole((2, HP, 4 * HP)),
                whole((2, 2 * HP, 4 * HP)),
                whole((2, 1, 4 * HP)),
                whole((2, HP, 4 * HP)),
                whole((2 * HP, kp)),
                whole((1, kp)),
                whole((1, kp)),
                whole((1, kp)),
                whole((k8, kp)),
            ],
            out_specs=whole((b_full, t_total)),
            scratch_shapes=[
                pltpu.VMEM((2, t_total // 2, b_full, 4 * HP), jnp.bfloat16),
                pltpu.VMEM((t_total, b_full, 2 * HP), jnp.bfloat16),
                pltpu.VMEM((t_total, b_full, 2 * HP), jnp.bfloat16),
                pltpu.VMEM((t_total, b_full, kp), jnp.float32),
                pltpu.VMEM((t_total, b_full, kp), jnp.bfloat16),
                pltpu.VMEM((b_full, HP), jnp.float32),
                pltpu.VMEM((b_full, HP), jnp.float32),
                pltpu.VMEM((b_full, HP), jnp.float32),
                pltpu.VMEM((b_full, HP), jnp.float32),
            ],
        ),
        compiler_params=pltpu.CompilerParams(
            dimension_semantics=("arbitrary",)),
    )(x_emb, len_col,
      stack2(layer0_w_in), stack2(layer0_b_in), layer0_whh,
      stack2(layer1_w_in), stack2(layer1_b_in), layer1_whh,
      w_out, b_out, crf_start, crf_end, crf_trans)
    return tags_bt.T


# speedup vs baseline: 1.8232x; 1.0080x over previous
import functools

import jax
import jax.numpy as jnp
from jax import lax
from jax.experimental import pallas as pl
from jax.experimental.pallas import tpu as pltpu

HP = 256
GEMM_CHUNK = 16
LSTM_UNROLL = 8
VIT_UNROLL = 8


def _lstm_cell(g_in, whh, h_prev, c_prev, m):
    gates = g_in + jnp.dot(h_prev.astype(jnp.bfloat16), whh,
                           preferred_element_type=jnp.float32)
    ig = jax.nn.sigmoid(gates[:, 0 * HP:1 * HP])
    fg = jax.nn.sigmoid(gates[:, 1 * HP:2 * HP])
    gg = jnp.tanh(gates[:, 2 * HP:3 * HP])
    og = jax.nn.sigmoid(gates[:, 3 * HP:4 * HP])
    c_new = fg * c_prev + ig * gg
    h_new = og * jnp.tanh(c_new)
    c_out = jnp.where(m, c_new, c_prev)
    h_out = jnp.where(m, h_new, h_prev)
    return h_new, h_out, c_out


def _inproj_half(src_ref, w_ref, b_ref, g_scr, phase, t_total, bp):
    din = src_ref.shape[-1]
    th = t_total // 2
    for d in range(2):
        w_in = w_ref[d]
        bias = b_ref[d]
        base = (phase if d == 0 else 1 - phase) * th
        for i in range(th // GEMM_CHUNK):
            xb = src_ref[base + i * GEMM_CHUNK:base + (i + 1) * GEMM_CHUNK]
            x2d = xb.reshape(GEMM_CHUNK * bp, din).astype(jnp.bfloat16)
            acc = jnp.dot(x2d, w_in,
                          preferred_element_type=jnp.float32) + bias
            g_scr[d, i * GEMM_CHUNK:(i + 1) * GEMM_CHUNK] = (
                acc.astype(jnp.bfloat16).reshape(GEMM_CHUNK, bp, 4 * HP))


def _bilstm_layer(src_ref, w_ref, b_ref, lenf, whh_ref, out_scr, g_scr,
                  hf_scr, cf_scr, hb_scr, cb_scr, t_total, bp, unroll):
    hf_scr[...] = jnp.zeros_like(hf_scr)
    cf_scr[...] = jnp.zeros_like(cf_scr)
    hb_scr[...] = jnp.zeros_like(hb_scr)
    cb_scr[...] = jnp.zeros_like(cb_scr)
    whh_f = whh_ref[0]
    whh_b = whh_ref[1]
    th = t_total // 2

    for phase in range(2):
        _inproj_half(src_ref, w_ref, b_ref, g_scr, phase, t_total, bp)
        f_base = phase * th
        b_base = (1 - phase) * th

        def step(s, carry):
            tf = s
            tb = t_total - 1 - s
            g_f = g_scr[0, tf - f_base].astype(jnp.float32)
            g_b = g_scr[1, tb - b_base].astype(jnp.float32)
            mf = lenf > tf
            mb = lenf > tb
            hf_new, hf_out, cf_out = _lstm_cell(g_f, whh_f, hf_scr[...],
                                                cf_scr[...], mf)
            hb_new, hb_out, cb_out = _lstm_cell(g_b, whh_b, hb_scr[...],
                                                cb_scr[...], mb)
            hf_scr[...] = hf_out
            cf_scr[...] = cf_out
            hb_scr[...] = hb_out
            cb_scr[...] = cb_out
            out_scr[tf, :, 0:HP] = jnp.where(mf, hf_new, 0.0).astype(out_scr.dtype)
            out_scr[tb, :, HP:2 * HP] = jnp.where(mb, hb_new, 0.0).astype(out_scr.dtype)
            return carry

        lax.fori_loop(phase * th, (phase + 1) * th, step, 0, unroll=unroll)


def _net_kernel(x_ref, len_ref,
                w0_ref, b0_ref, whh0_ref,
                w1_ref, b1_ref, whh1_ref,
                wout_ref, bout_ref, start_ref, end_ref, trans_ref,
                tags_ref,
                g_scr, h0_scr, h1_scr, em_scr, hist_scr,
                hf_scr, cf_scr, hb_scr, cb_scr,
                *, t_total, unroll, vunroll):
    bp = x_ref.shape[1]
    lenf = len_ref[...]

    _bilstm_layer(x_ref, w0_ref, b0_ref, lenf, whh0_ref, h0_scr, g_scr,
                  hf_scr, cf_scr, hb_scr, cb_scr, t_total, bp, unroll)
    _bilstm_layer(h0_scr, w1_ref, b1_ref, lenf, whh1_ref, h1_scr, g_scr,
                  hf_scr, cf_scr, hb_scr, cb_scr, t_total, bp, unroll)

    d2 = h1_scr.shape[-1]
    k8, kp = trans_ref.shape
    for i in range(t_total // GEMM_CHUNK):
        hb = h1_scr[i * GEMM_CHUNK:(i + 1) * GEMM_CHUNK]
        em = (jnp.dot(hb.reshape(GEMM_CHUNK * bp, d2), wout_ref[...],
                      preferred_element_type=jnp.float32) + bout_ref[...])
        em_scr[i * GEMM_CHUNK:(i + 1) * GEMM_CHUNK] = (
            em.reshape(GEMM_CHUNK, bp, kp))

    trans8 = trans_ref[...]

    def fstep(t, score):
        em = em_scr[t]
        m = lenf > t
        prev = score
        best = prev[:, 0:1] + trans8[0:1, :]
        bidx = jnp.zeros_like(best)
        for j in range(1, k8):
            c = prev[:, j:j + 1] + trans8[j:j + 1, :]
            better = c > best
            best = jnp.where(better, c, best)
            bidx = jnp.where(better, float(j), bidx)
        upd = jnp.where(m, best + em, prev)
        nxt = jnp.where(t == 0, prev + em, upd)
        hist_scr[t] = jnp.where(t == 0, 0.0, bidx).astype(jnp.bfloat16)
        return nxt

    score0 = jnp.broadcast_to(start_ref[...], (bp, kp))
    score = lax.fori_loop(0, t_total, fstep, score0, unroll=vunroll)
    score = score + end_ref[...]

    lane = lax.broadcasted_iota(jnp.int32, (bp, kp), 1).astype(jnp.float32)
    maxv = jnp.max(score, axis=1, keepdims=True)
    best_last = jnp.min(jnp.where(score == maxv, lane, float(kp)),
                        axis=1, keepdims=True)
    seq_end = lenf - 1.0
    tlane = lax.broadcasted_iota(jnp.int32, (bp, t_total), 1).astype(jnp.float32)

    def bstep(s, carry):
        tags_acc, cur = carry
        t = t_total - 1 - s
        h = hist_scr[jnp.minimum(t + 1, t_total - 1)].astype(jnp.float32)
        picked = jnp.sum(jnp.where(lane == cur, h, 0.0), axis=1, keepdims=True)
        tag_t = jnp.where(t == seq_end, best_last,
                          jnp.where(t < seq_end, picked, 0.0))
        cur = jnp.where(t <= seq_end, tag_t, cur)
        tags_acc = jnp.where(tlane == t, tag_t, tags_acc)
        return (tags_acc, cur)

    tags0 = jnp.zeros((bp, t_total), jnp.float32)
    tags_acc, _ = lax.fori_loop(0, t_total, bstep, (tags0, best_last),
                                unroll=vunroll)
    tags_ref[...] = tags_acc.astype(jnp.int32)


def kernel(pad_index, embedding, w_out, b_out, crf_start, crf_end, crf_trans,
           layer0_w_in, layer0_b_in, layer0_whh,
           layer1_w_in, layer1_b_in, layer1_whh,
           x_ids, sent_lengths):
    t_total, b_full = x_ids.shape
    k8, kp = crf_trans.shape
    len_col = sent_lengths.astype(jnp.float32)[:, None]
    x_emb = embedding[x_ids.astype(jnp.int32)].astype(jnp.bfloat16)

    def stack2(w):
        return jnp.stack([w[:, :4 * HP], w[:, 4 * HP:]])

    kern = functools.partial(_net_kernel, t_total=t_total,
                             unroll=LSTM_UNROLL, vunroll=VIT_UNROLL)
    whole = lambda shape: pl.BlockSpec(shape, lambda i: (0,) * len(shape))
    emb_dim = x_emb.shape[-1]
    tags_bt = pl.pallas_call(
        kern,
        out_shape=jax.ShapeDtypeStruct((b_full, t_total), jnp.int32),
        grid_spec=pltpu.PrefetchScalarGridSpec(
            num_scalar_prefetch=0,
            grid=(1,),
            in_specs=[
                whole((t_total, b_full, emb_dim)),
                whole((b_full, 1)),
                whole((2, emb_dim, 4 * HP)),
                whole((2, 1, 4 * HP)),
                whole((2, HP, 4 * HP)),
                whole((2, 2 * HP, 4 * HP)),
                whole((2, 1, 4 * HP)),
                whole((2, HP, 4 * HP)),
                whole((2 * HP, kp)),
                whole((1, kp)),
                whole((1, kp)),
                whole((1, kp)),
                whole((k8, kp)),
            ],
            out_specs=whole((b_full, t_total)),
            scratch_shapes=[
                pltpu.VMEM((2, t_total // 2, b_full, 4 * HP), jnp.bfloat16),
                pltpu.VMEM((t_total, b_full, 2 * HP), jnp.bfloat16),
                pltpu.VMEM((t_total, b_full, 2 * HP), jnp.bfloat16),
                pltpu.VMEM((t_total, b_full, kp), jnp.float32),
                pltpu.VMEM((t_total, b_full, kp), jnp.bfloat16),
                pltpu.VMEM((b_full, HP), jnp.float32),
                pltpu.VMEM((b_full, HP), jnp.float32),
                pltpu.VMEM((b_full, HP), jnp.float32),
                pltpu.VMEM((b_full, HP), jnp.float32),
            ],
        ),
        compiler_params=pltpu.CompilerParams(
            dimension_semantics=("arbitrary",)),
    )(x_emb, len_col,
      stack2(layer0_w_in), stack2(layer0_b_in), layer0_whh,
      stack2(layer1_w_in), stack2(layer1_b_in), layer1_whh,
      w_out, b_out, crf_start, crf_end, crf_trans)
    return tags_bt.T


# speedup vs baseline: 1.8595x; 1.0199x over previous
import functools

import jax
import jax.numpy as jnp
from jax import lax
from jax.experimental import pallas as pl
from jax.experimental.pallas import tpu as pltpu

HP = 256
GEMM_CHUNK = 16
LSTM_UNROLL = 16
VIT_UNROLL = 16


def _lstm_cell(g_in, whh, h_prev, c_prev, m):
    gates = g_in + jnp.dot(h_prev.astype(jnp.bfloat16), whh,
                           preferred_element_type=jnp.float32)
    ig = jax.nn.sigmoid(gates[:, 0 * HP:1 * HP])
    fg = jax.nn.sigmoid(gates[:, 1 * HP:2 * HP])
    gg = jnp.tanh(gates[:, 2 * HP:3 * HP])
    og = jax.nn.sigmoid(gates[:, 3 * HP:4 * HP])
    c_new = fg * c_prev + ig * gg
    h_new = og * jnp.tanh(c_new)
    c_out = jnp.where(m, c_new, c_prev)
    h_out = jnp.where(m, h_new, h_prev)
    return h_new, h_out, c_out


def _inproj_half(src_ref, w_ref, b_ref, g_scr, phase, t_total, bp):
    din = src_ref.shape[-1]
    th = t_total // 2
    for d in range(2):
        w_in = w_ref[d]
        bias = b_ref[d]
        base = (phase if d == 0 else 1 - phase) * th
        for i in range(th // GEMM_CHUNK):
            xb = src_ref[base + i * GEMM_CHUNK:base + (i + 1) * GEMM_CHUNK]
            x2d = xb.reshape(GEMM_CHUNK * bp, din).astype(jnp.bfloat16)
            acc = jnp.dot(x2d, w_in,
                          preferred_element_type=jnp.float32) + bias
            g_scr[d, i * GEMM_CHUNK:(i + 1) * GEMM_CHUNK] = (
                acc.astype(jnp.bfloat16).reshape(GEMM_CHUNK, bp, 4 * HP))


def _bilstm_layer(src_ref, w_ref, b_ref, lenf, whh_ref, out_scr, g_scr,
                  hf_scr, cf_scr, hb_scr, cb_scr, t_total, bp, unroll):
    hf_scr[...] = jnp.zeros_like(hf_scr)
    cf_scr[...] = jnp.zeros_like(cf_scr)
    hb_scr[...] = jnp.zeros_like(hb_scr)
    cb_scr[...] = jnp.zeros_like(cb_scr)
    whh_f = whh_ref[0]
    whh_b = whh_ref[1]
    th = t_total // 2

    for phase in range(2):
        _inproj_half(src_ref, w_ref, b_ref, g_scr, phase, t_total, bp)
        f_base = phase * th
        b_base = (1 - phase) * th

        def step(s, carry):
            tf = s
            tb = t_total - 1 - s
            g_f = g_scr[0, tf - f_base].astype(jnp.float32)
            g_b = g_scr[1, tb - b_base].astype(jnp.float32)
            mf = lenf > tf
            mb = lenf > tb
            hf_new, hf_out, cf_out = _lstm_cell(g_f, whh_f, hf_scr[...],
                                                cf_scr[...], mf)
            hb_new, hb_out, cb_out = _lstm_cell(g_b, whh_b, hb_scr[...],
                                                cb_scr[...], mb)
            hf_scr[...] = hf_out
            cf_scr[...] = cf_out
            hb_scr[...] = hb_out
            cb_scr[...] = cb_out
            out_scr[tf, :, 0:HP] = jnp.where(mf, hf_new, 0.0).astype(out_scr.dtype)
            out_scr[tb, :, HP:2 * HP] = jnp.where(mb, hb_new, 0.0).astype(out_scr.dtype)
            return carry

        lax.fori_loop(phase * th, (phase + 1) * th, step, 0, unroll=unroll)


def _net_kernel(x_ref, len_ref,
                w0_ref, b0_ref, whh0_ref,
                w1_ref, b1_ref, whh1_ref,
                wout_ref, bout_ref, start_ref, end_ref, trans_ref,
                tags_ref,
                g_scr, h0_scr, h1_scr, em_scr, hist_scr,
                hf_scr, cf_scr, hb_scr, cb_scr,
                *, t_total, unroll, vunroll):
    bp = x_ref.shape[1]
    lenf = len_ref[...]

    _bilstm_layer(x_ref, w0_ref, b0_ref, lenf, whh0_ref, h0_scr, g_scr,
                  hf_scr, cf_scr, hb_scr, cb_scr, t_total, bp, unroll)
    _bilstm_layer(h0_scr, w1_ref, b1_ref, lenf, whh1_ref, h1_scr, g_scr,
                  hf_scr, cf_scr, hb_scr, cb_scr, t_total, bp, unroll)

    d2 = h1_scr.shape[-1]
    k8, kp = trans_ref.shape
    for i in range(t_total // GEMM_CHUNK):
        hb = h1_scr[i * GEMM_CHUNK:(i + 1) * GEMM_CHUNK]
        em = (jnp.dot(hb.reshape(GEMM_CHUNK * bp, d2), wout_ref[...],
                      preferred_element_type=jnp.float32) + bout_ref[...])
        em_scr[i * GEMM_CHUNK:(i + 1) * GEMM_CHUNK] = (
            em.reshape(GEMM_CHUNK, bp, kp))

    trans8 = trans_ref[...]

    def fstep(t, score):
        em = em_scr[t]
        m = lenf > t
        prev = score
        best = prev[:, 0:1] + trans8[0:1, :]
        bidx = jnp.zeros_like(best)
        for j in range(1, k8):
            c = prev[:, j:j + 1] + trans8[j:j + 1, :]
            better = c > best
            best = jnp.where(better, c, best)
            bidx = jnp.where(better, float(j), bidx)
        upd = jnp.where(m, best + em, prev)
        nxt = jnp.where(t == 0, prev + em, upd)
        hist_scr[t] = jnp.where(t == 0, 0.0, bidx).astype(jnp.bfloat16)
        return nxt

    score0 = jnp.broadcast_to(start_ref[...], (bp, kp))
    score = lax.fori_loop(0, t_total, fstep, score0, unroll=vunroll)
    score = score + end_ref[...]

    lane = lax.broadcasted_iota(jnp.int32, (bp, kp), 1).astype(jnp.float32)
    maxv = jnp.max(score, axis=1, keepdims=True)
    best_last = jnp.min(jnp.where(score == maxv, lane, float(kp)),
                        axis=1, keepdims=True)
    seq_end = lenf - 1.0
    tlane = lax.broadcasted_iota(jnp.int32, (bp, t_total), 1).astype(jnp.float32)

    def bstep(s, carry):
        tags_acc, cur = carry
        t = t_total - 1 - s
        h = hist_scr[jnp.minimum(t + 1, t_total - 1)].astype(jnp.float32)
        picked = jnp.sum(jnp.where(lane == cur, h, 0.0), axis=1, keepdims=True)
        tag_t = jnp.where(t == seq_end, best_last,
                          jnp.where(t < seq_end, picked, 0.0))
        cur = jnp.where(t <= seq_end, tag_t, cur)
        tags_acc = jnp.where(tlane == t, tag_t, tags_acc)
        return (tags_acc, cur)

    tags0 = jnp.zeros((bp, t_total), jnp.float32)
    tags_acc, _ = lax.fori_loop(0, t_total, bstep, (tags0, best_last),
                                unroll=vunroll)
    tags_ref[...] = tags_acc.astype(jnp.int32)


def kernel(pad_index, embedding, w_out, b_out, crf_start, crf_end, crf_trans,
           layer0_w_in, layer0_b_in, layer0_whh,
           layer1_w_in, layer1_b_in, layer1_whh,
           x_ids, sent_lengths):
    t_total, b_full = x_ids.shape
    k8, kp = crf_trans.shape
    len_col = sent_lengths.astype(jnp.float32)[:, None]
    x_emb = embedding[x_ids.astype(jnp.int32)].astype(jnp.bfloat16)

    def stack2(w):
        return jnp.stack([w[:, :4 * HP], w[:, 4 * HP:]])

    kern = functools.partial(_net_kernel, t_total=t_total,
                             unroll=LSTM_UNROLL, vunroll=VIT_UNROLL)
    whole = lambda shape: pl.BlockSpec(shape, lambda i: (0,) * len(shape))
    emb_dim = x_emb.shape[-1]
    tags_bt = pl.pallas_call(
        kern,
        out_shape=jax.ShapeDtypeStruct((b_full, t_total), jnp.int32),
        grid_spec=pltpu.PrefetchScalarGridSpec(
            num_scalar_prefetch=0,
            grid=(1,),
            in_specs=[
                whole((t_total, b_full, emb_dim)),
                whole((b_full, 1)),
                whole((2, emb_dim, 4 * HP)),
                whole((2, 1, 4 * HP)),
                whole((2, HP, 4 * HP)),
                whole((2, 2 * HP, 4 * HP)),
                whole((2, 1, 4 * HP)),
                whole((2, HP, 4 * HP)),
                whole((2 * HP, kp)),
                whole((1, kp)),
                whole((1, kp)),
                whole((1, kp)),
                whole((k8, kp)),
            ],
            out_specs=whole((b_full, t_total)),
            scratch_shapes=[
                pltpu.VMEM((2, t_total // 2, b_full, 4 * HP), jnp.bfloat16),
                pltpu.VMEM((t_total, b_full, 2 * HP), jnp.bfloat16),
                pltpu.VMEM((t_total, b_full, 2 * HP), jnp.bfloat16),
                pltpu.VMEM((t_total, b_full, kp), jnp.float32),
                pltpu.VMEM((t_total, b_full, kp), jnp.bfloat16),
                pltpu.VMEM((b_full, HP), jnp.float32),
                pltpu.VMEM((b_full, HP), jnp.float32),
                pltpu.VMEM((b_full, HP), jnp.float32),
                pltpu.VMEM((b_full, HP), jnp.float32),
            ],
        ),
        compiler_params=pltpu.CompilerParams(
            dimension_semantics=("arbitrary",)),
    )(x_emb, len_col,
      stack2(layer0_w_in), stack2(layer0_b_in), layer0_whh,
      stack2(layer1_w_in), stack2(layer1_b_in), layer1_whh,
      w_out, b_out, crf_start, crf_end, crf_trans)
    return tags_bt.T


# speedup vs baseline: 1.8784x; 1.0101x over previous
import functools

import jax
import jax.numpy as jnp
from jax import lax
from jax.experimental import pallas as pl
from jax.experimental.pallas import tpu as pltpu

HP = 256
GEMM_CHUNK = 16
LSTM_UNROLL = 16
VIT_UNROLL = 16


def _lstm_cell(g_in, whh, h_prev, c_prev, m):
    gates = g_in + jnp.dot(h_prev.astype(jnp.bfloat16), whh,
                           preferred_element_type=jnp.float32)
    ig = jax.nn.sigmoid(gates[:, 0 * HP:1 * HP])
    fg = jax.nn.sigmoid(gates[:, 1 * HP:2 * HP])
    gg = jnp.tanh(gates[:, 2 * HP:3 * HP])
    og = jax.nn.sigmoid(gates[:, 3 * HP:4 * HP])
    c_new = fg * c_prev + ig * gg
    h_new = og * jnp.tanh(c_new)
    c_out = jnp.where(m, c_new, c_prev)
    h_out = jnp.where(m, h_new, h_prev)
    return h_new, h_out, c_out


def _inproj_half(src_ref, w_ref, b_ref, g_scr, phase, t_total, bp):
    din = src_ref.shape[-1]
    th = t_total // 2
    for d in range(2):
        w_in = w_ref[d]
        bias = b_ref[d]
        base = (phase if d == 0 else 1 - phase) * th
        for i in range(th // GEMM_CHUNK):
            xb = src_ref[base + i * GEMM_CHUNK:base + (i + 1) * GEMM_CHUNK]
            x2d = xb.reshape(GEMM_CHUNK * bp, din).astype(jnp.bfloat16)
            acc = jnp.dot(x2d, w_in,
                          preferred_element_type=jnp.float32) + bias
            g_scr[d, i * GEMM_CHUNK:(i + 1) * GEMM_CHUNK] = (
                acc.astype(jnp.bfloat16).reshape(GEMM_CHUNK, bp, 4 * HP))


def _bilstm_layer(src_ref, w_ref, b_ref, lenf, whh_ref, out_scr, g_scr,
                  hf_scr, cf_scr, hb_scr, cb_scr, t_total, bp, unroll):
    hf_scr[...] = jnp.zeros_like(hf_scr)
    cf_scr[...] = jnp.zeros_like(cf_scr)
    hb_scr[...] = jnp.zeros_like(hb_scr)
    cb_scr[...] = jnp.zeros_like(cb_scr)
    whh_f = whh_ref[0]
    whh_b = whh_ref[1]
    th = t_total // 2

    for phase in range(2):
        _inproj_half(src_ref, w_ref, b_ref, g_scr, phase, t_total, bp)
        f_base = phase * th
        b_base = (1 - phase) * th

        def step(s, carry):
            tf = s
            tb = t_total - 1 - s
            g_f = g_scr[0, tf - f_base].astype(jnp.float32)
            g_b = g_scr[1, tb - b_base].astype(jnp.float32)
            mf = lenf > tf
            mb = lenf > tb
            hf_new, hf_out, cf_out = _lstm_cell(g_f, whh_f, hf_scr[...],
                                                cf_scr[...], mf)
            hb_new, hb_out, cb_out = _lstm_cell(g_b, whh_b, hb_scr[...],
                                                cb_scr[...], mb)
            hf_scr[...] = hf_out
            cf_scr[...] = cf_out
            hb_scr[...] = hb_out
            cb_scr[...] = cb_out
            out_scr[tf, :, 0:HP] = jnp.where(mf, hf_new, 0.0).astype(out_scr.dtype)
            out_scr[tb, :, HP:2 * HP] = jnp.where(mb, hb_new, 0.0).astype(out_scr.dtype)
            return carry

        lax.fori_loop(phase * th, (phase + 1) * th, step, 0, unroll=unroll)


def _net_kernel(x_ref, len_ref,
                w0_ref, b0_ref, whh0_ref,
                w1_ref, b1_ref, whh1_ref,
                wout_ref, bout_ref, start_ref, end_ref, trans_ref,
                tags_ref,
                g_scr, h0_scr, h1_scr, em_scr, hist_scr,
                hf_scr, cf_scr, hb_scr, cb_scr,
                *, t_total, unroll, vunroll):
    bp = x_ref.shape[1]
    lenf = len_ref[...]

    _bilstm_layer(x_ref, w0_ref, b0_ref, lenf, whh0_ref, h0_scr, g_scr,
                  hf_scr, cf_scr, hb_scr, cb_scr, t_total, bp, unroll)
    _bilstm_layer(h0_scr, w1_ref, b1_ref, lenf, whh1_ref, h1_scr, g_scr,
                  hf_scr, cf_scr, hb_scr, cb_scr, t_total, bp, unroll)

    d2 = h1_scr.shape[-1]
    k8, kp = trans_ref.shape
    for i in range(t_total // GEMM_CHUNK):
        hb = h1_scr[i * GEMM_CHUNK:(i + 1) * GEMM_CHUNK]
        em = (jnp.dot(hb.reshape(GEMM_CHUNK * bp, d2), wout_ref[...],
                      preferred_element_type=jnp.float32) + bout_ref[...])
        em_scr[i * GEMM_CHUNK:(i + 1) * GEMM_CHUNK] = (
            em.reshape(GEMM_CHUNK, bp, kp))

    trans8 = trans_ref[...]

    def fstep(t, score):
        em = em_scr[t]
        m = lenf > t
        prev = score
        nodes = []
        for j in range(k8):
            nodes.append((prev[:, j:j + 1] + trans8[j:j + 1, :], float(j)))
        while len(nodes) > 1:
            nxt_nodes = []
            for a in range(0, len(nodes) - 1, 2):
                v1, i1 = nodes[a]
                v2, i2 = nodes[a + 1]
                take2 = v2 > v1
                v = jnp.where(take2, v2, v1)
                if isinstance(i1, float):
                    i = jnp.where(take2, i2, jnp.full_like(v, i1))
                else:
                    i = jnp.where(take2, i2, i1)
                nxt_nodes.append((v, i))
            if len(nodes) % 2:
                nxt_nodes.append(nodes[-1])
            nodes = nxt_nodes
        best, bidx = nodes[0]
        if isinstance(bidx, float):
            bidx = jnp.full_like(best, bidx)
        upd = jnp.where(m, best + em, prev)
        nxt = jnp.where(t == 0, prev + em, upd)
        hist_scr[t] = jnp.where(t == 0, 0.0, bidx).astype(jnp.bfloat16)
        return nxt

    score0 = jnp.broadcast_to(start_ref[...], (bp, kp))
    score = lax.fori_loop(0, t_total, fstep, score0, unroll=vunroll)
    score = score + end_ref[...]

    lane = lax.broadcasted_iota(jnp.int32, (bp, kp), 1).astype(jnp.float32)
    maxv = jnp.max(score, axis=1, keepdims=True)
    best_last = jnp.min(jnp.where(score == maxv, lane, float(kp)),
                        axis=1, keepdims=True)
    seq_end = lenf - 1.0
    tlane = lax.broadcasted_iota(jnp.int32, (bp, t_total), 1).astype(jnp.float32)

    def bstep(s, carry):
        tags_acc, cur = carry
        t = t_total - 1 - s
        h = hist_scr[jnp.minimum(t + 1, t_total - 1)].astype(jnp.float32)
        picked = jnp.sum(jnp.where(lane == cur, h, 0.0), axis=1, keepdims=True)
        tag_t = jnp.where(t == seq_end, best_last,
                          jnp.where(t < seq_end, picked, 0.0))
        cur = jnp.where(t <= seq_end, tag_t, cur)
        tags_acc = jnp.where(tlane == t, tag_t, tags_acc)
        return (tags_acc, cur)

    tags0 = jnp.zeros((bp, t_total), jnp.float32)
    tags_acc, _ = lax.fori_loop(0, t_total, bstep, (tags0, best_last),
                                unroll=vunroll)
    tags_ref[...] = tags_acc.astype(jnp.int32)


def kernel(pad_index, embedding, w_out, b_out, crf_start, crf_end, crf_trans,
           layer0_w_in, layer0_b_in, layer0_whh,
           layer1_w_in, layer1_b_in, layer1_whh,
           x_ids, sent_lengths):
    t_total, b_full = x_ids.shape
    k8, kp = crf_trans.shape
    len_col = sent_lengths.astype(jnp.float32)[:, None]
    x_emb = embedding[x_ids.astype(jnp.int32)].astype(jnp.bfloat16)

    def stack2(w):
        return jnp.stack([w[:, :4 * HP], w[:, 4 * HP:]])

    kern = functools.partial(_net_kernel, t_total=t_total,
                             unroll=LSTM_UNROLL, vunroll=VIT_UNROLL)
    whole = lambda shape: pl.BlockSpec(shape, lambda i: (0,) * len(shape))
    emb_dim = x_emb.shape[-1]
    tags_bt = pl.pallas_call(
        kern,
        out_shape=jax.ShapeDtypeStruct((b_full, t_total), jnp.int32),
        grid_spec=pltpu.PrefetchScalarGridSpec(
            num_scalar_prefetch=0,
            grid=(1,),
            in_specs=[
                whole((t_total, b_full, emb_dim)),
                whole((b_full, 1)),
                whole((2, emb_dim, 4 * HP)),
                whole((2, 1, 4 * HP)),
                whole((2, HP, 4 * HP)),
                whole((2, 2 * HP, 4 * HP)),
                whole((2, 1, 4 * HP)),
                whole((2, HP, 4 * HP)),
                whole((2 * HP, kp)),
                whole((1, kp)),
                whole((1, kp)),
                whole((1, kp)),
                whole((k8, kp)),
            ],
            out_specs=whole((b_full, t_total)),
            scratch_shapes=[
                pltpu.VMEM((2, t_total // 2, b_full, 4 * HP), jnp.bfloat16),
                pltpu.VMEM((t_total, b_full, 2 * HP), jnp.bfloat16),
                pltpu.VMEM((t_total, b_full, 2 * HP), jnp.bfloat16),
                pltpu.VMEM((t_total, b_full, kp), jnp.float32),
                pltpu.VMEM((t_total, b_full, kp), jnp.bfloat16),
                pltpu.VMEM((b_full, HP), jnp.float32),
                pltpu.VMEM((b_full, HP), jnp.float32),
                pltpu.VMEM((b_full, HP), jnp.float32),
                pltpu.VMEM((b_full, HP), jnp.float32),
            ],
        ),
        compiler_params=pltpu.CompilerParams(
            dimension_semantics=("arbitrary",)),
    )(x_emb, len_col,
      stack2(layer0_w_in), stack2(layer0_b_in), layer0_whh,
      stack2(layer1_w_in), stack2(layer1_b_in), layer1_whh,
      w_out, b_out, crf_start, crf_end, crf_trans)
    return tags_bt.T
